# baseline (speedup 1.0000x reference)
import jax
import jax.numpy as jnp
from jax import lax
from jax.experimental import pallas as pl
from jax.experimental.pallas import tpu as pltpu
from jax.experimental.pallas import tpu_sc as plsc

_NUM_CORES = 2
_NUM_SUBCORES = 16
_NW = _NUM_CORES * _NUM_SUBCORES
_B = 16384
_D = 16
_C = 1000000
_LAMBDA_C = 0.003

_CW = 2048
_KPW = 16
_SHARD = _CW * _KPW
_NFULL = _C // _CW
_TAIL_LO = _NFULL * _CW
_TAIL_COL0 = 1000064 - _CW
_FB = 128
_NVEC = _B // 16


def _cl_body(featR_hbm, lbl_hbm, cent_hbm, out_hbm,
             lbl_v, bidx_v, cm_v, cb0_v, cb1_v, gi_v, fgrp_v, acc_v, sem):
    wid = lax.axis_index("s") * _NUM_CORES + lax.axis_index("c")
    lane = lax.iota(jnp.int32, 16)
    zero16 = jnp.zeros((16,), jnp.float32)

    pltpu.sync_copy(lbl_hbm, lbl_v)

    shard_lo = wid * _SHARD
    shard_hi = shard_lo + _SHARD

    def scan_step(i, off):
        v = lbl_v[pl.ds(i * 16, 16)]
        m = (v >= shard_lo) & (v < shard_hi)
        plsc.store_compressed(bidx_v.at[pl.ds(off, 16)], i * 16 + lane, mask=m)
        return off + jnp.max(plsc.all_reduce_population_count(m))

    nm = lax.fori_loop(0, _NVEC, scan_step, jnp.int32(0))
    nmv = (nm + 15) // 16

    def process(acc, col0, flt_lo, flt_hi):
        def refil(i, off):
            g = i * 16 + lane
            bv = jnp.clip(bidx_v[pl.ds(i * 16, 16)], 0, _B - 1)
            cls = plsc.load_gather(lbl_v, [bv])
            m = (cls >= flt_lo) & (cls < flt_hi) & (g < nm)
            plsc.store_compressed(cm_v.at[pl.ds(off, 16)], bv, mask=m)
            return off + jnp.max(plsc.all_reduce_population_count(m))

        ncm = lax.fori_loop(0, nmv, refil, jnp.int32(0))
        nf = (ncm + _FB - 1) // _FB

        def fblock(j, acc):
            for v in range(_FB // 16):
                bv = jnp.clip(cm_v[pl.ds(j * _FB + v * 16, 16)], 0, _B - 1)
                gi_v[pl.ds(v * 16, 16)] = lax.shift_right_logical(bv, 3)
            pltpu.async_copy(featR_hbm.at[gi_v], fgrp_v, sem).wait()
            for v in range(_FB // 16):
                base = j * _FB + v * 16
                bv = jnp.clip(cm_v[pl.ds(base, 16)], 0, _B - 1)
                cls = plsc.load_gather(lbl_v, [bv])
                col = jnp.clip(cls - col0, 0, _CW - 1)
                m = (base + lane) < ncm
                rowv = v * 16 + lane
                fcol = (bv & 7) * 16
                for d in range(_D):
                    cb = cb0_v if d < 8 else cb1_v
                    sub = jnp.full((16,), d % 8, jnp.int32)
                    cw = plsc.load_gather(cb, [sub, col])
                    fw = plsc.load_gather(fgrp_v, [rowv, fcol + d])
                    diff = jnp.where(m, fw - cw, zero16)
                    acc = acc + diff * diff
            return acc

        return lax.fori_loop(0, nf, fblock, acc)

    acc = zero16
    acc_v[...] = zero16

    def chunk_step(k, carry):
        c = wid * _KPW + k

        @pl.when(c < _NFULL)
        def _full():
            col0 = pl.multiple_of(c * _CW, _CW)
            pltpu.sync_copy(cent_hbm.at[0, :, pl.ds(col0, _CW)], cb0_v)
            pltpu.sync_copy(cent_hbm.at[1, :, pl.ds(col0, _CW)], cb1_v)

        @pl.when(c == _NFULL)
        def _tail():
            pltpu.sync_copy(cent_hbm.at[0, :, pl.ds(_TAIL_LO, 512)],
                            cb0_v.at[:, pl.ds(0, 512)])
            pltpu.sync_copy(cent_hbm.at[1, :, pl.ds(_TAIL_LO, 512)],
                            cb1_v.at[:, pl.ds(0, 512)])
            for f in range(8):
                pltpu.sync_copy(cent_hbm.at[0, f, pl.ds(_TAIL_LO + 512, 64)],
                                cb0_v.at[f, pl.ds(512, 64)])
                pltpu.sync_copy(cent_hbm.at[1, f, pl.ds(_TAIL_LO + 512, 64)],
                                cb1_v.at[f, pl.ds(512, 64)])

        is_full = c < _NFULL
        col0 = jnp.where(is_full, c * _CW, _TAIL_LO)
        flt_lo = jnp.where(is_full, c * _CW, _TAIL_LO)
        flt_hi = jnp.where(is_full, (c + 1) * _CW, jnp.int32(_C))

        @pl.when(c <= _NFULL)
        def _proc():
            acc2 = process(acc, col0, flt_lo, flt_hi)
            acc_v[...] = acc_v[...] + acc2

        return carry

    lax.fori_loop(0, _KPW, chunk_step, jnp.int32(0))
    pltpu.sync_copy(acc_v, out_hbm.at[pl.ds(wid * 16, 16)])


@jax.jit
def kernel(features, labels, centers):
    lbl = labels.reshape(_B)
    featR = features.reshape(_B * _D // 128, 128)
    cent = centers.T.reshape(_NUM_CORES, 8, _C)
    mesh = plsc.VectorSubcoreMesh(core_axis_name="c", subcore_axis_name="s")
    partials = pl.kernel(
        _cl_body,
        out_type=jax.ShapeDtypeStruct((_NW * _D,), jnp.float32),
        mesh=mesh,
        scratch_types=[
            pltpu.VMEM((_B,), jnp.int32),
            pltpu.VMEM((_B,), jnp.int32),
            pltpu.VMEM((_B,), jnp.int32),
            pltpu.VMEM((8, _CW), jnp.float32),
            pltpu.VMEM((8, _CW), jnp.float32),
            pltpu.VMEM((_FB,), jnp.int32),
            pltpu.VMEM((_FB, 128), jnp.float32),
            pltpu.VMEM((_D,), jnp.float32),
            pltpu.SemaphoreType.DMA,
        ],
        compiler_params=pltpu.CompilerParams(needs_layout_passes=False),
    )(featR, lbl, cent)
    return _LAMBDA_C * (jnp.sum(partials) / 2.0 / _B)


# speedup vs baseline: 4.2008x; 4.2008x over previous
import jax
import jax.numpy as jnp
from jax import lax
from jax.experimental import pallas as pl
from jax.experimental.pallas import tpu as pltpu
from jax.experimental.pallas import tpu_sc as plsc

_NUM_CORES = 2
_NUM_SUBCORES = 16
_NW = _NUM_CORES * _NUM_SUBCORES
_B = 16384
_D = 16
_C = 1000000
_LAMBDA_C = 0.003

_CW = 4096
_KPW = 8
_SHARD = _CW * _KPW
_NFULL = _C // _CW
_TAIL_LO = _NFULL * _CW
_TAIL_COL0 = 1000064 - _CW
_FB = 112
_NVEC = _B // 16


def _cl_body(featR_hbm, lbl_hbm, cent_hbm, out_hbm,
             lbl_v, bidx_v, cm_v, cb0_v, cb1_v, gi_v, fgrp_v, acc_v, sem):
    wid = lax.axis_index("s") * _NUM_CORES + lax.axis_index("c")
    lane = lax.iota(jnp.int32, 16)
    zero16 = jnp.zeros((16,), jnp.float32)

    pltpu.sync_copy(lbl_hbm, lbl_v)

    shard_lo = wid * _SHARD
    shard_hi = shard_lo + _SHARD

    def scan_step(i, off):
        v = lbl_v[pl.ds(i * 16, 16)]
        m = (v >= shard_lo) & (v < shard_hi)
        plsc.store_compressed(bidx_v.at[pl.ds(off, 16)], i * 16 + lane, mask=m)
        return off + jnp.max(plsc.all_reduce_population_count(m))

    nm = lax.fori_loop(0, _NVEC, scan_step, jnp.int32(0))
    nmv = (nm + 15) // 16

    def process(acc, col0, flt_lo, flt_hi):
        def refil(i, off):
            g = i * 16 + lane
            bv = jnp.clip(bidx_v[pl.ds(i * 16, 16)], 0, _B - 1)
            cls = plsc.load_gather(lbl_v, [bv])
            m = (cls >= flt_lo) & (cls < flt_hi) & (g < nm)
            plsc.store_compressed(cm_v.at[pl.ds(off, 16)], bv, mask=m)
            return off + jnp.max(plsc.all_reduce_population_count(m))

        ncm = lax.fori_loop(0, nmv, refil, jnp.int32(0))
        nf = (ncm + _FB - 1) // _FB

        def fblock(j, acc):
            for v in range(_FB // 16):
                bv = jnp.clip(cm_v[pl.ds(j * _FB + v * 16, 16)], 0, _B - 1)
                gi_v[pl.ds(v * 16, 16)] = lax.shift_right_logical(bv, 3)
            pltpu.async_copy(featR_hbm.at[gi_v], fgrp_v, sem).wait()
            for v in range(_FB // 16):
                base = j * _FB + v * 16
                bv = jnp.clip(cm_v[pl.ds(base, 16)], 0, _B - 1)
                cls = plsc.load_gather(lbl_v, [bv])
                col = jnp.clip(cls - col0, 0, _CW - 1)
                m = (base + lane) < ncm
                rowv = v * 16 + lane
                fcol = (bv & 7) * 16
                for d in range(_D):
                    cb = cb0_v if d < 8 else cb1_v
                    sub = jnp.full((16,), d % 8, jnp.int32)
                    cw = plsc.load_gather(cb, [sub, col])
                    fw = plsc.load_gather(fgrp_v, [rowv, fcol + d])
                    diff = jnp.where(m, fw - cw, zero16)
                    acc = acc + diff * diff
            return acc

        return lax.fori_loop(0, nf, fblock, acc)

    acc = zero16
    acc_v[...] = zero16
    for k in range(_KPW):
        c = wid * _KPW + k

        @pl.when(c < _NFULL)
        def _full():
            col0 = pl.multiple_of(c * _CW, _CW)
            pltpu.sync_copy(cent_hbm.at[0, :, pl.ds(col0, _CW)], cb0_v)
            pltpu.sync_copy(cent_hbm.at[1, :, pl.ds(col0, _CW)], cb1_v)

        @pl.when(c == _NFULL)
        def _tail():
            pltpu.sync_copy(cent_hbm.at[0, :, pl.ds(_TAIL_LO, 512)],
                            cb0_v.at[:, pl.ds(0, 512)])
            pltpu.sync_copy(cent_hbm.at[1, :, pl.ds(_TAIL_LO, 512)],
                            cb1_v.at[:, pl.ds(0, 512)])
            for f in range(8):
                pltpu.sync_copy(cent_hbm.at[0, f, pl.ds(_TAIL_LO + 512, 64)],
                                cb0_v.at[f, pl.ds(512, 64)])
                pltpu.sync_copy(cent_hbm.at[1, f, pl.ds(_TAIL_LO + 512, 64)],
                                cb1_v.at[f, pl.ds(512, 64)])

        is_full = c < _NFULL
        col0 = jnp.where(is_full, c * _CW, _TAIL_LO)
        flt_lo = jnp.where(is_full, c * _CW, _TAIL_LO)
        flt_hi = jnp.where(is_full, (c + 1) * _CW, jnp.int32(_C))

        @pl.when(c <= _NFULL)
        def _proc():
            acc2 = process(acc, col0, flt_lo, flt_hi)
            acc_v[...] = acc_v[...] + acc2

    pltpu.sync_copy(acc_v, out_hbm.at[pl.ds(wid * 16, 16)])


@jax.jit
def kernel(features, labels, centers):
    lbl = labels.reshape(_B)
    featR = features.reshape(_B * _D // 128, 128)
    cent = centers.T.reshape(_NUM_CORES, 8, _C)
    mesh = plsc.VectorSubcoreMesh(core_axis_name="c", subcore_axis_name="s")
    partials = pl.kernel(
        _cl_body,
        out_type=jax.ShapeDtypeStruct((_NW * _D,), jnp.float32),
        mesh=mesh,
        scratch_types=[
            pltpu.VMEM((_B,), jnp.int32),
            pltpu.VMEM((_B,), jnp.int32),
            pltpu.VMEM((_B,), jnp.int32),
            pltpu.VMEM((8, _CW), jnp.float32),
            pltpu.VMEM((8, _CW), jnp.float32),
            pltpu.VMEM((_FB,), jnp.int32),
            pltpu.VMEM((_FB, 128), jnp.float32),
            pltpu.VMEM((_D,), jnp.float32),
            pltpu.SemaphoreType.DMA,
        ],
        compiler_params=pltpu.CompilerParams(needs_layout_passes=False),
    )(featR, lbl, cent)
    return _LAMBDA_C * (jnp.sum(partials) / 2.0 / _B)
